# baseline (speedup 1.0000x reference)
import functools

import jax
import jax.numpy as jnp
from jax import lax
from jax.experimental import pallas as pl
from jax.experimental.pallas import tpu as pltpu
from jax.experimental.pallas import tpu_sc as plsc


@functools.lru_cache(maxsize=None)
def _build(M, D, B):
  NC = 2
  NS = 16
  NT = NC * NS
  L = 16
  SCH = 1024
  OWN = (M // NT) // 8 * 8
  LAST_EXTRA = M - NT * OWN
  NCOPY = 4
  CRS = [OWN // NCOPY // 8 * 8] * (NCOPY - 1)
  CRS.append(OWN - sum(CRS))

  assert B % SCH == 0 and D % L == 0 and OWN % 8 == 0
  assert LAST_EXTRA >= 0 and LAST_EXTRA % 8 == 0 and all(c % 8 == 0 for c in CRS)

  mesh = plsc.VectorSubcoreMesh(core_axis_name="c", subcore_axis_name="s")

  @functools.partial(
      pl.kernel,
      out_type=jax.ShapeDtypeStruct((M, D), jnp.float32),
      mesh=mesh,
      scratch_types=[
          pltpu.VMEM((B,), jnp.int32),
          pltpu.VMEM((8, D), jnp.float32),
          pltpu.VMEM((8, D), jnp.float32),
          pltpu.SemaphoreType.DMA,
          pltpu.SemaphoreType.DMA,
      ],
      compiler_params=pltpu.CompilerParams(use_tc_tiling_on_sc=False),
  )
  def run(mem_h, idx_h, val_h, out_h, idx_v, rowb, valb, sem, csem):
    c = lax.axis_index("c")
    s = lax.axis_index("s")
    t = c * NS + s
    lo = t * OWN
    hi = lo + OWN + LAST_EXTRA * (t == NT - 1)

    off = 0
    cps = []
    for cr in CRS:
      cps.append(pltpu.async_copy(mem_h.at[pl.ds(lo + off, cr)],
                                  out_h.at[pl.ds(lo + off, cr)], csem))
      off += cr
    if LAST_EXTRA:
      @pl.when(t == NT - 1)
      def _extra_copy():
        pltpu.async_copy(mem_h.at[pl.ds(NT * OWN, LAST_EXTRA)],
                         out_h.at[pl.ds(NT * OWN, LAST_EXTRA)], csem)
    for cp in cps:
      cp.wait()
    if LAST_EXTRA:
      @pl.when(t == NT - 1)
      def _extra_drain():
        pltpu.make_async_copy(mem_h.at[pl.ds(NT * OWN, LAST_EXTRA)],
                              out_h.at[pl.ds(NT * OWN, LAST_EXTRA)],
                              csem).wait()

    pltpu.sync_copy(idx_h, idx_v)

    def body(g, carry):
      vec = idx_v[pl.ds(g * L, L)]

      for i in range(L):
        row = vec[i]

        @pl.when((row >= lo) & (row < hi))
        def _apply(row=row, i=i):
          b8 = (row >> 3) << 3
          o8 = row - b8
          gj = g * L + i
          vb8 = (gj >> 3) << 3
          vo8 = gj - vb8
          cp1 = pltpu.async_copy(out_h.at[pl.ds(b8, 8)], rowb, sem)
          cp2 = pltpu.async_copy(val_h.at[pl.ds(vb8, 8)], valb, sem)
          cp1.wait()
          cp2.wait()
          for k in range(D // L):
            rowb[o8, pl.ds(k * L, L)] = (rowb[o8, pl.ds(k * L, L)] +
                                         valb[vo8, pl.ds(k * L, L)])
          pltpu.async_copy(rowb, out_h.at[pl.ds(b8, 8)], sem).wait()

      return carry

    lax.fori_loop(0, B // L, body, 0)

  return run


def kernel(mem, idx, val):
  M, D = mem.shape
  B = idx.shape[0]
  return _build(M, D, B)(mem, idx.astype(jnp.int32), val)
